# baseline (speedup 1.0000x reference)
import functools

import jax
import jax.numpy as jnp
from jax import lax
from jax.experimental import pallas as pl
from jax.experimental.pallas import tpu as pltpu
from jax.experimental.pallas import tpu_sc as plsc

_NUM_WORKERS = 32


def kernel(x, pe_table):
    b, size, dx = x.shape
    dim = pe_table.shape[-1]
    rows = size // _NUM_WORKERS
    mesh = plsc.VectorSubcoreMesh(core_axis_name="c", subcore_axis_name="s")

    @functools.partial(
        pl.kernel,
        mesh=mesh,
        out_type=jax.ShapeDtypeStruct((b, size, dx + dim), x.dtype),
        scratch_types=[pltpu.MemorySpace.VMEM((rows, dx + dim), x.dtype)],
    )
    def run(x_hbm, pe_hbm, out_hbm, buf):
        wid = lax.axis_index("s") * 2 + lax.axis_index("c")
        s0 = wid * rows
        pltpu.sync_copy(pe_hbm.at[pl.ds(s0, rows), :], buf.at[:, dx:])
        for bb in range(b):
            pltpu.sync_copy(x_hbm.at[bb, pl.ds(s0, rows), :], buf.at[:, :dx])
            pltpu.sync_copy(buf, out_hbm.at[bb, pl.ds(s0, rows), :])

    return run(x, pe_table)


# speedup vs baseline: 1.0980x; 1.0980x over previous
import functools

import jax
import jax.numpy as jnp
from jax import lax
from jax.experimental import pallas as pl
from jax.experimental.pallas import tpu as pltpu
from jax.experimental.pallas import tpu_sc as plsc

_NUM_WORKERS = 32


def kernel(x, pe_table):
    b, size, dx = x.shape
    dim = pe_table.shape[-1]
    rows = size // _NUM_WORKERS
    mesh = plsc.VectorSubcoreMesh(core_axis_name="c", subcore_axis_name="s")

    @functools.partial(
        pl.kernel,
        mesh=mesh,
        out_type=jax.ShapeDtypeStruct((b, size, dx + dim), x.dtype),
        scratch_types=[
            pltpu.MemorySpace.VMEM((rows, dim), x.dtype),
            pltpu.MemorySpace.VMEM((2, rows, dx), x.dtype),
            pltpu.SemaphoreType.DMA,
            pltpu.SemaphoreType.DMA((2,)),
            pltpu.SemaphoreType.DMA((2,)),
            pltpu.SemaphoreType.DMA,
        ],
    )
    def run(x_hbm, pe_hbm, out_hbm, pebuf, xbuf, sem_pe, sem_xl, sem_xs, sem_ps):
        wid = lax.axis_index("s") * 2 + lax.axis_index("c")
        s0 = wid * rows
        pe_load = pltpu.make_async_copy(pe_hbm.at[pl.ds(s0, rows), :], pebuf, sem_pe)
        pe_load.start()
        x_loads = [
            pltpu.make_async_copy(
                x_hbm.at[bb, pl.ds(s0, rows), :], xbuf.at[bb % 2], sem_xl.at[bb % 2]
            )
            for bb in range(b)
        ]
        x_stores = [
            pltpu.make_async_copy(
                xbuf.at[bb % 2],
                out_hbm.at[bb, pl.ds(s0, rows), pl.ds(0, dx)],
                sem_xs.at[bb % 2],
            )
            for bb in range(b)
        ]
        pe_stores = [
            pltpu.make_async_copy(
                pebuf, out_hbm.at[bb, pl.ds(s0, rows), pl.ds(dx, dim)], sem_ps
            )
            for bb in range(b)
        ]
        x_loads[0].start()
        if b > 1:
            x_loads[1].start()
        pe_load.wait()
        for bb in range(b):
            x_loads[bb].wait()
            x_stores[bb].start()
            pe_stores[bb].start()
            if bb + 2 < b:
                x_stores[bb].wait()
                x_loads[bb + 2].start()
        for bb in range(max(0, b - 2), b):
            x_stores[bb].wait()
        for bb in range(b):
            pe_stores[bb].wait()

    return run(x, pe_table)
